# baseline (speedup 1.0000x reference)
import functools

import jax
import jax.numpy as jnp
from jax import lax
from jax.experimental import pallas as pl
from jax.experimental.pallas import tpu as pltpu
from jax.experimental.pallas import tpu_sc as plsc

KK = 15
DD = 3
EPSG = 1e-15
EDGE_BLK = 2048

CHUNK = 128
NC, NS = 2, 16
NW = NC * NS
NJ = 40
EP = NW * NJ * CHUNK
ZROWS = 1000
FW = 128
DUMP = 240


def _sc_mesh():
    return plsc.VectorSubcoreMesh(core_axis_name="c", subcore_axis_name="s")


def _gather_body(h_hbm, src_hbm, out_hbm, idx2, rows_v, sem):
    wid = lax.axis_index("s") * NC + lax.axis_index("c")
    base = wid * NJ
    pltpu.sync_copy(src_hbm.at[pl.ds(base, NJ)], idx2)

    def step(t, carry):
        pltpu.async_copy(h_hbm.at[idx2.at[t]], rows_v, sem).wait()
        pltpu.sync_copy(rows_v, out_hbm.at[pl.ds((base + t) * CHUNK, CHUNK)])
        return carry

    lax.fori_loop(0, NJ, step, 0)


def _sc_gather(h, src2):
    n, fw = h.shape
    k = pl.kernel(
        _gather_body,
        out_type=jax.ShapeDtypeStruct((EP, fw), jnp.float32),
        mesh=_sc_mesh(),
        scratch_types=[
            pltpu.VMEM((NJ, CHUNK), jnp.int32),
            pltpu.VMEM((CHUNK, fw), jnp.float32),
            pltpu.SemaphoreType.DMA,
        ],
    )
    return k(h, src2)


def _scatter_body(n, do_cnt, msg_hbm, dst_hbm, zero_hbm, zero1_hbm, *refs):
    if do_cnt:
        (parts_hbm, cnt_hbm, idx2, vals0, vals1, ones_v,
         semv0, semv1, sems0, sems1, acc_s, cnt_s) = refs
    else:
        (parts_hbm, idx2, vals0, vals1,
         semv0, semv1, sems0, sems1, acc_s) = refs
        cnt_s = None
    cid = lax.axis_index("c")
    sid = lax.axis_index("s")
    wid = sid * NC + cid
    base = wid * NJ
    vals = (vals0, vals1)
    semv = (semv0, semv1)
    sems = (sems0, sems1)
    nslab = n // ZROWS

    @pl.when(sid < nslab)
    def _():
        pltpu.sync_copy(zero_hbm, acc_s.at[pl.ds(sid * ZROWS, ZROWS)])

    if do_cnt:
        ncs = cnt_s.shape[0] // 1024

        @pl.when(sid == nslab)
        def _():
            for t in range(ncs):
                pltpu.sync_copy(zero1_hbm, cnt_s.at[pl.ds(t * 1024, 1024)])

        for t in range(CHUNK // 16):
            ones_v[pl.ds(t * 16, 16)] = jnp.ones((16,), jnp.float32)
    plsc.subcore_barrier()

    pltpu.sync_copy(dst_hbm.at[pl.ds(base, NJ)], idx2)
    pltpu.async_copy(msg_hbm.at[pl.ds(base * CHUNK, CHUNK)], vals0, semv0)

    def step(u, carry):
        for b in range(2):
            t = 2 * u + b
            pltpu.make_async_copy(
                msg_hbm.at[pl.ds((base + t) * CHUNK, CHUNK)], vals[b],
                semv[b]).wait()
            pltpu.async_copy(vals[b], acc_s.at[idx2.at[t]], sems[b], add=True)
            if do_cnt:
                pltpu.sync_copy(ones_v, cnt_s.at[idx2.at[t]], add=True)

            @pl.when(t + 1 < NJ)
            def _():
                @pl.when(t >= 1)
                def _():
                    pltpu.make_async_copy(
                        vals[1 - b], acc_s.at[idx2.at[t - 1]],
                        sems[1 - b]).wait()
                pltpu.async_copy(
                    msg_hbm.at[pl.ds((base + t + 1) * CHUNK, CHUNK)],
                    vals[1 - b], semv[1 - b])

        return carry

    lax.fori_loop(0, NJ // 2, step, 0)
    pltpu.make_async_copy(vals0, acc_s.at[idx2.at[NJ - 2]], sems0).wait()
    pltpu.make_async_copy(vals1, acc_s.at[idx2.at[NJ - 1]], sems1).wait()
    plsc.subcore_barrier()

    @pl.when(sid < nslab)
    def _():
        rows = pl.ds(sid * ZROWS, ZROWS)
        pltpu.sync_copy(acc_s.at[rows], parts_hbm.at[cid].at[rows])

    if do_cnt:
        ncs = cnt_s.shape[0] // 1024

        @pl.when(sid == nslab)
        def _():
            for t in range(ncs):
                rows = pl.ds(t * 1024, 1024)
                pltpu.sync_copy(cnt_s.at[rows], cnt_hbm.at[cid].at[rows])


def _sc_scatter(msg, dst2, n, do_cnt):
    e, f_out = msg.shape
    npad = ((n + DUMP + 1023) // 1024) * 1024
    zero = jnp.zeros((ZROWS, f_out), jnp.float32)
    zero1 = jnp.zeros((1024,), jnp.float32)
    out_type = [jax.ShapeDtypeStruct((NC, n, f_out), jnp.float32)]
    scratch = [
        pltpu.VMEM((NJ, CHUNK), jnp.int32),
        pltpu.VMEM((CHUNK, f_out), jnp.float32),
        pltpu.VMEM((CHUNK, f_out), jnp.float32),
    ]
    if do_cnt:
        out_type.append(jax.ShapeDtypeStruct((NC, npad), jnp.float32))
        scratch.append(pltpu.VMEM((CHUNK,), jnp.float32))
    scratch += [
        pltpu.SemaphoreType.DMA,
        pltpu.SemaphoreType.DMA,
        pltpu.SemaphoreType.DMA,
        pltpu.SemaphoreType.DMA,
    ]
    scratch.append(pltpu.VMEM_SHARED((n + DUMP, f_out), jnp.float32))
    if do_cnt:
        scratch.append(pltpu.VMEM_SHARED((npad,), jnp.float32))
    k = pl.kernel(
        functools.partial(_scatter_body, n, do_cnt),
        out_type=tuple(out_type) if do_cnt else out_type[0],
        mesh=_sc_mesh(),
        scratch_types=scratch,
    )
    return k(msg, dst2, zero, zero1)


def _edge_body(attr_ref, xe_ref, mu_ref, alpha_ref, g_ref, msg_ref, z_ref):
    eb = attr_ref.shape[0]
    acc_g = jnp.zeros((eb, KK), dtype=jnp.float32)
    for d in range(DD):
        col = attr_ref[:, d : d + 1]
        diff = col - mu_ref[d : d + 1, :]
        acc_g = acc_g + diff * diff * alpha_ref[d : d + 1, :]
    gauss = jnp.exp(acc_g).astype(jnp.bfloat16)
    xe = xe_ref[...].astype(jnp.bfloat16)
    for k in range(KK):
        z_ref[:, k * FW : (k + 1) * FW] = xe * gauss[:, k : k + 1]
    msg_ref[...] = jnp.dot(z_ref[...], g_ref[...],
                           preferred_element_type=jnp.float32)


def _edge_msgs(edge_attr, xe, mu_t, alpha_t, g3):
    e = xe.shape[0]
    f_in = xe.shape[1]
    f_out = g3.shape[2]
    g_flat = g3.reshape(KK * f_in, f_out).astype(jnp.bfloat16)
    assert e % EDGE_BLK == 0
    grid = e // EDGE_BLK
    return pl.pallas_call(
        _edge_body,
        grid=(grid,),
        in_specs=[
            pl.BlockSpec((EDGE_BLK, DD), lambda i: (i, 0)),
            pl.BlockSpec((EDGE_BLK, f_in), lambda i: (i, 0)),
            pl.BlockSpec((DD, KK), lambda i: (0, 0)),
            pl.BlockSpec((DD, KK), lambda i: (0, 0)),
            pl.BlockSpec((KK * f_in, f_out), lambda i: (0, 0)),
        ],
        out_specs=pl.BlockSpec((EDGE_BLK, f_out), lambda i: (i, 0)),
        out_shape=jax.ShapeDtypeStruct((e, f_out), jnp.float32),
        scratch_shapes=[pltpu.VMEM((EDGE_BLK, KK * f_in), jnp.bfloat16)],
    )(edge_attr, xe, mu_t, alpha_t, g_flat)


def _node_body(do_act, parts_ref, cnt_ref, x_ref, root_ref, bias_ref,
               gamma_ref, beta_ref, out_ref):
    agg = (parts_ref[0] + parts_ref[1]) / jnp.maximum(cnt_ref[...], 1.0)
    r = jnp.dot(x_ref[...], root_ref[...], preferred_element_type=jnp.float32)
    h = agg + r + bias_ref[...]
    if do_act:
        h = jnp.where(h > 0, h, jnp.exp(h) - 1.0)
        m = jnp.mean(h, axis=0, keepdims=True)
        c = h - m
        v = jnp.mean(c * c, axis=0, keepdims=True)
        h = c / jnp.sqrt(v + 1e-5) * gamma_ref[...] + beta_ref[...]
    out_ref[...] = h


def _node_update(parts, cnt, x, root, bias, gamma, beta, do_act):
    n = x.shape[0]
    f_out = root.shape[1]
    return pl.pallas_call(
        functools.partial(_node_body, do_act),
        out_shape=jax.ShapeDtypeStruct((n, f_out), jnp.float32),
    )(parts, cnt, x, root, bias, gamma, beta)


def _padw(a, w=FW):
    pad = [(0, 0)] * (a.ndim - 1) + [(0, w - a.shape[-1])]
    return jnp.pad(a, pad)


def kernel(x, edge_index, edge_attr, params):
    n = x.shape[0]
    e = edge_attr.shape[0]
    f_final = params["conv4"]["root"].shape[1]
    assert e <= EP
    src = edge_index[0].astype(jnp.int32)
    dst = edge_index[1].astype(jnp.int32)
    src2 = jnp.pad(src, (0, EP - e)).reshape(EP // CHUNK, CHUNK)
    dst_pad = n + jnp.arange(EP - e, dtype=jnp.int32) % DUMP
    dst2 = jnp.concatenate([dst, dst_pad]).reshape(EP // CHUNK, CHUNK)
    attr_p = jnp.pad(edge_attr, ((0, EP - e), (0, 0)))
    h = _padw(x)
    cnt = None
    names = ("conv1", "conv2", "conv3", "conv4")
    bns = ("bn1", "bn2", "bn3", None)
    for name, bn in zip(names, bns):
        p = params[name]
        f_in, f_out = p["root"].shape
        mu_t = p["mu"].T
        alpha_t = (-0.5 / (EPSG + p["sigma"] ** 2)).T
        g3 = p["g"].reshape(f_in, KK, f_out).transpose(1, 0, 2)
        g3 = jnp.pad(g3, ((0, 0), (0, FW - f_in), (0, FW - f_out)))
        xe = _sc_gather(h, src2)
        msg = _edge_msgs(attr_p, xe, mu_t, alpha_t, g3)
        if cnt is None:
            parts, cnt2 = _sc_scatter(msg, dst2, n, do_cnt=True)
            cnt = (cnt2[0, :n] + cnt2[1, :n]).reshape(n, 1)
        else:
            parts = _sc_scatter(msg, dst2, n, do_cnt=False)
        if bn is None:
            gamma = jnp.ones((1, FW), jnp.float32)
            beta = jnp.zeros((1, FW), jnp.float32)
        else:
            gamma = _padw(params[bn]["gamma"].reshape(1, f_out))
            beta = _padw(params[bn]["beta"].reshape(1, f_out))
        h = _node_update(parts, cnt, h, jnp.pad(p["root"], ((0, FW - f_in), (0, FW - f_out))),
                         _padw(p["bias"].reshape(1, f_out)), gamma, beta,
                         do_act=bn is not None)
    return h[:, :f_final]


# speedup vs baseline: 1.3784x; 1.3784x over previous
import functools

import jax
import jax.numpy as jnp
from jax import lax
from jax.experimental import pallas as pl
from jax.experimental.pallas import tpu as pltpu
from jax.experimental.pallas import tpu_sc as plsc

KK = 15
DD = 3
EPSG = 1e-15
EDGE_BLK = 2048

CHUNK = 128
NC, NS = 2, 16
NW = NC * NS
NJ = 40
EP = NW * NJ * CHUNK
ZROWS = 1000
FW = 128
DUMP = 240


def _sc_mesh():
    return plsc.VectorSubcoreMesh(core_axis_name="c", subcore_axis_name="s")


def _gather_body(h_hbm, src_hbm, out_hbm, idx2, rows_v, sem):
    wid = lax.axis_index("s") * NC + lax.axis_index("c")
    base = wid * NJ
    pltpu.sync_copy(src_hbm.at[pl.ds(base, NJ)], idx2)

    def step(t, carry):
        pltpu.async_copy(h_hbm.at[idx2.at[t]], rows_v, sem).wait()
        pltpu.sync_copy(rows_v, out_hbm.at[pl.ds((base + t) * CHUNK, CHUNK)])
        return carry

    lax.fori_loop(0, NJ, step, 0)


def _sc_gather(h, src2):
    n, fw = h.shape
    k = pl.kernel(
        _gather_body,
        out_type=jax.ShapeDtypeStruct((EP, fw), jnp.float32),
        mesh=_sc_mesh(),
        scratch_types=[
            pltpu.VMEM((NJ, CHUNK), jnp.int32),
            pltpu.VMEM((CHUNK, fw), jnp.float32),
            pltpu.SemaphoreType.DMA,
        ],
    )
    return k(h, src2)


def _scatter_body(n, do_cnt, msg_hbm, dst_hbm, zero_hbm, zero1_hbm, *refs):
    if do_cnt:
        (parts_hbm, cnt_hbm, idx2, vals0, vals1, ones_v,
         semv0, semv1, sems0, sems1, acc_s, cnt_s) = refs
    else:
        (parts_hbm, idx2, vals0, vals1,
         semv0, semv1, sems0, sems1, acc_s) = refs
        cnt_s = None
    cid = lax.axis_index("c")
    sid = lax.axis_index("s")
    wid = sid * NC + cid
    base = wid * NJ
    vals = (vals0, vals1)
    semv = (semv0, semv1)
    sems = (sems0, sems1)
    nslab = n // ZROWS

    @pl.when(sid < nslab)
    def _():
        pltpu.sync_copy(zero_hbm, acc_s.at[pl.ds(sid * ZROWS, ZROWS)])

    if do_cnt:
        ncs = cnt_s.shape[0] // 1024

        @pl.when(sid == nslab)
        def _():
            for t in range(ncs):
                pltpu.sync_copy(zero1_hbm, cnt_s.at[pl.ds(t * 1024, 1024)])

        for t in range(CHUNK // 16):
            ones_v[pl.ds(t * 16, 16)] = jnp.ones((16,), jnp.float32)
    plsc.subcore_barrier()

    pltpu.sync_copy(dst_hbm.at[pl.ds(base, NJ)], idx2)
    pltpu.async_copy(msg_hbm.at[pl.ds(base * CHUNK, CHUNK)], vals0, semv0)

    def step(u, carry):
        for b in range(2):
            t = 2 * u + b
            pltpu.make_async_copy(
                msg_hbm.at[pl.ds((base + t) * CHUNK, CHUNK)], vals[b],
                semv[b]).wait()
            pltpu.async_copy(vals[b], acc_s.at[idx2.at[t]], sems[b], add=True)
            if do_cnt:
                pltpu.sync_copy(ones_v, cnt_s.at[idx2.at[t]], add=True)

            @pl.when(t + 1 < NJ)
            def _():
                @pl.when(t >= 1)
                def _():
                    pltpu.make_async_copy(
                        vals[1 - b], acc_s.at[idx2.at[t - 1]],
                        sems[1 - b]).wait()
                pltpu.async_copy(
                    msg_hbm.at[pl.ds((base + t + 1) * CHUNK, CHUNK)],
                    vals[1 - b], semv[1 - b])

        return carry

    lax.fori_loop(0, NJ // 2, step, 0)
    pltpu.make_async_copy(vals0, acc_s.at[idx2.at[NJ - 2]], sems0).wait()
    pltpu.make_async_copy(vals1, acc_s.at[idx2.at[NJ - 1]], sems1).wait()
    plsc.subcore_barrier()

    @pl.when(sid < nslab)
    def _():
        rows = pl.ds(sid * ZROWS, ZROWS)
        pltpu.sync_copy(acc_s.at[rows], parts_hbm.at[cid].at[rows])

    if do_cnt:
        ncs = cnt_s.shape[0] // 1024

        @pl.when(sid == nslab)
        def _():
            for t in range(ncs):
                rows = pl.ds(t * 1024, 1024)
                pltpu.sync_copy(cnt_s.at[rows], cnt_hbm.at[cid].at[rows])


def _sc_scatter(msg, dst2, n, do_cnt):
    e, f_out = msg.shape
    npad = ((n + DUMP + 1023) // 1024) * 1024
    zero = jnp.zeros((ZROWS, f_out), jnp.float32)
    zero1 = jnp.zeros((1024,), jnp.float32)
    out_type = [jax.ShapeDtypeStruct((NC, n, f_out), jnp.float32)]
    scratch = [
        pltpu.VMEM((NJ, CHUNK), jnp.int32),
        pltpu.VMEM((CHUNK, f_out), jnp.float32),
        pltpu.VMEM((CHUNK, f_out), jnp.float32),
    ]
    if do_cnt:
        out_type.append(jax.ShapeDtypeStruct((NC, npad), jnp.float32))
        scratch.append(pltpu.VMEM((CHUNK,), jnp.float32))
    scratch += [
        pltpu.SemaphoreType.DMA,
        pltpu.SemaphoreType.DMA,
        pltpu.SemaphoreType.DMA,
        pltpu.SemaphoreType.DMA,
    ]
    scratch.append(pltpu.VMEM_SHARED((n + DUMP, f_out), jnp.float32))
    if do_cnt:
        scratch.append(pltpu.VMEM_SHARED((npad,), jnp.float32))
    k = pl.kernel(
        functools.partial(_scatter_body, n, do_cnt),
        out_type=tuple(out_type) if do_cnt else out_type[0],
        mesh=_sc_mesh(),
        scratch_types=scratch,
    )
    return k(msg, dst2, zero, zero1)


def _edge_body(attr_ref, xe_ref, mu_ref, alpha_ref, g_ref, msg_ref, z_ref):
    eb = attr_ref.shape[0]
    acc_g = jnp.zeros((eb, KK), dtype=jnp.float32)
    for d in range(DD):
        col = attr_ref[:, d : d + 1]
        diff = col - mu_ref[d : d + 1, :]
        acc_g = acc_g + diff * diff * alpha_ref[d : d + 1, :]
    gauss = jnp.exp(acc_g).astype(jnp.bfloat16)
    xe = xe_ref[...].astype(jnp.bfloat16)
    for k in range(KK):
        z_ref[:, k * FW : (k + 1) * FW] = xe * gauss[:, k : k + 1]
    msg_ref[...] = jnp.dot(z_ref[...], g_ref[...],
                           preferred_element_type=jnp.float32)


def _edge_msgs(edge_attr, xe, mu_t, alpha_t, g3):
    e = xe.shape[0]
    f_in = xe.shape[1]
    f_out = g3.shape[2]
    g_flat = g3.reshape(KK * f_in, f_out).astype(jnp.bfloat16)
    assert e % EDGE_BLK == 0
    grid = e // EDGE_BLK
    return pl.pallas_call(
        _edge_body,
        grid=(grid,),
        in_specs=[
            pl.BlockSpec((EDGE_BLK, DD), lambda i: (i, 0)),
            pl.BlockSpec((EDGE_BLK, f_in), lambda i: (i, 0)),
            pl.BlockSpec((DD, KK), lambda i: (0, 0)),
            pl.BlockSpec((DD, KK), lambda i: (0, 0)),
            pl.BlockSpec((KK * f_in, f_out), lambda i: (0, 0)),
        ],
        out_specs=pl.BlockSpec((EDGE_BLK, f_out), lambda i: (i, 0)),
        out_shape=jax.ShapeDtypeStruct((e, f_out), jnp.float32),
        scratch_shapes=[pltpu.VMEM((EDGE_BLK, KK * f_in), jnp.bfloat16)],
    )(edge_attr, xe, mu_t, alpha_t, g_flat)


def _node_body(do_act, parts_ref, cnt_ref, x_ref, root_ref, bias_ref,
               gamma_ref, beta_ref, out_ref):
    agg = (parts_ref[0] + parts_ref[1]) / jnp.maximum(cnt_ref[...], 1.0)
    r = jnp.dot(x_ref[...], root_ref[...], preferred_element_type=jnp.float32)
    h = agg + r + bias_ref[...]
    if do_act:
        h = jnp.where(h > 0, h, jnp.exp(h) - 1.0)
        m = jnp.mean(h, axis=0, keepdims=True)
        c = h - m
        v = jnp.mean(c * c, axis=0, keepdims=True)
        h = c / jnp.sqrt(v + 1e-5) * gamma_ref[...] + beta_ref[...]
    out_ref[...] = h


def _node_update(parts, cnt, x, root, bias, gamma, beta, do_act):
    n = x.shape[0]
    f_out = root.shape[1]
    return pl.pallas_call(
        functools.partial(_node_body, do_act),
        out_shape=jax.ShapeDtypeStruct((n, f_out), jnp.float32),
    )(parts, cnt, x, root, bias, gamma, beta)


def _padw(a, w=FW):
    pad = [(0, 0)] * (a.ndim - 1) + [(0, w - a.shape[-1])]
    return jnp.pad(a, pad)


def kernel(x, edge_index, edge_attr, params):
    n = x.shape[0]
    e = edge_attr.shape[0]
    f_final = params["conv4"]["root"].shape[1]
    assert e <= EP
    src = edge_index[0].astype(jnp.int32)
    dst = edge_index[1].astype(jnp.int32)
    src_pad = jnp.arange(EP - e, dtype=jnp.int32) % n
    src2 = jnp.concatenate([src, src_pad]).reshape(EP // CHUNK, CHUNK)
    dst_pad = n + jnp.arange(EP - e, dtype=jnp.int32) % DUMP
    dst2 = jnp.concatenate([dst, dst_pad]).reshape(EP // CHUNK, CHUNK)
    attr_p = jnp.pad(edge_attr, ((0, EP - e), (0, 0)))
    h = _padw(x)
    cnt = None
    names = ("conv1", "conv2", "conv3", "conv4")
    bns = ("bn1", "bn2", "bn3", None)
    for name, bn in zip(names, bns):
        p = params[name]
        f_in, f_out = p["root"].shape
        mu_t = p["mu"].T
        alpha_t = (-0.5 / (EPSG + p["sigma"] ** 2)).T
        g3 = p["g"].reshape(f_in, KK, f_out).transpose(1, 0, 2)
        g3 = jnp.pad(g3, ((0, 0), (0, FW - f_in), (0, FW - f_out)))
        xe = _sc_gather(h, src2)
        msg = _edge_msgs(attr_p, xe, mu_t, alpha_t, g3)
        if cnt is None:
            parts, cnt2 = _sc_scatter(msg, dst2, n, do_cnt=True)
            cnt = (cnt2[0, :n] + cnt2[1, :n]).reshape(n, 1)
        else:
            parts = _sc_scatter(msg, dst2, n, do_cnt=False)
        if bn is None:
            gamma = jnp.ones((1, FW), jnp.float32)
            beta = jnp.zeros((1, FW), jnp.float32)
        else:
            gamma = _padw(params[bn]["gamma"].reshape(1, f_out))
            beta = _padw(params[bn]["beta"].reshape(1, f_out))
        h = _node_update(parts, cnt, h, jnp.pad(p["root"], ((0, FW - f_in), (0, FW - f_out))),
                         _padw(p["bias"].reshape(1, f_out)), gamma, beta,
                         do_act=bn is not None)
    return h[:, :f_final]


# speedup vs baseline: 1.4066x; 1.0204x over previous
import functools

import jax
import jax.numpy as jnp
from jax import lax
from jax.experimental import pallas as pl
from jax.experimental.pallas import tpu as pltpu
from jax.experimental.pallas import tpu_sc as plsc

KK = 15
DD = 3
EPSG = 1e-15
EDGE_BLK = 2048

CHUNK = 128
NC, NS = 2, 16
NW = NC * NS
NJ = 40
EP = NW * NJ * CHUNK
ZROWS = 1000
FW = 128
DUMP = 240


def _sc_mesh():
    return plsc.VectorSubcoreMesh(core_axis_name="c", subcore_axis_name="s")


def _gather_body(h_hbm, src_hbm, out_hbm, idx2, rows0, rows1,
                 semg0, semg1, semw0, semw1):
    wid = lax.axis_index("s") * NC + lax.axis_index("c")
    base = wid * NJ
    rows = (rows0, rows1)
    semg = (semg0, semg1)
    semw = (semw0, semw1)
    pltpu.sync_copy(src_hbm.at[pl.ds(base, NJ)], idx2)
    pltpu.async_copy(h_hbm.at[idx2.at[0]], rows0, semg0)

    def step(u, carry):
        for b in range(2):
            t = 2 * u + b
            pltpu.make_async_copy(h_hbm.at[idx2.at[t]], rows[b], semg[b]).wait()
            pltpu.async_copy(
                rows[b], out_hbm.at[pl.ds((base + t) * CHUNK, CHUNK)], semw[b])

            @pl.when(t + 1 < NJ)
            def _():
                @pl.when(t >= 1)
                def _():
                    pltpu.make_async_copy(
                        rows[1 - b],
                        out_hbm.at[pl.ds((base + t - 1) * CHUNK, CHUNK)],
                        semw[1 - b]).wait()
                pltpu.async_copy(h_hbm.at[idx2.at[t + 1]], rows[1 - b],
                                 semg[1 - b])

        return carry

    lax.fori_loop(0, NJ // 2, step, 0)
    pltpu.make_async_copy(
        rows0, out_hbm.at[pl.ds((base + NJ - 2) * CHUNK, CHUNK)], semw0).wait()
    pltpu.make_async_copy(
        rows1, out_hbm.at[pl.ds((base + NJ - 1) * CHUNK, CHUNK)], semw1).wait()


def _sc_gather(h, src2):
    n, fw = h.shape
    k = pl.kernel(
        _gather_body,
        out_type=jax.ShapeDtypeStruct((EP, fw), jnp.float32),
        mesh=_sc_mesh(),
        scratch_types=[
            pltpu.VMEM((NJ, CHUNK), jnp.int32),
            pltpu.VMEM((CHUNK, fw), jnp.float32),
            pltpu.VMEM((CHUNK, fw), jnp.float32),
            pltpu.SemaphoreType.DMA,
            pltpu.SemaphoreType.DMA,
            pltpu.SemaphoreType.DMA,
            pltpu.SemaphoreType.DMA,
        ],
    )
    return k(h, src2)


def _scatter_body(n, do_cnt, msg_hbm, dst_hbm, zero_hbm, zero1_hbm, *refs):
    if do_cnt:
        (parts_hbm, cnt_hbm, idx2, vals0, vals1, ones_v,
         semv0, semv1, sems0, sems1, acc_s, cnt_s) = refs
    else:
        (parts_hbm, idx2, vals0, vals1,
         semv0, semv1, sems0, sems1, acc_s) = refs
        cnt_s = None
    cid = lax.axis_index("c")
    sid = lax.axis_index("s")
    wid = sid * NC + cid
    base = wid * NJ
    vals = (vals0, vals1)
    semv = (semv0, semv1)
    sems = (sems0, sems1)
    nslab = n // ZROWS

    @pl.when(sid < nslab)
    def _():
        pltpu.sync_copy(zero_hbm, acc_s.at[pl.ds(sid * ZROWS, ZROWS)])

    if do_cnt:
        ncs = cnt_s.shape[0] // 1024

        @pl.when(sid == nslab)
        def _():
            for t in range(ncs):
                pltpu.sync_copy(zero1_hbm, cnt_s.at[pl.ds(t * 1024, 1024)])

        for t in range(CHUNK // 16):
            ones_v[pl.ds(t * 16, 16)] = jnp.ones((16,), jnp.float32)
    plsc.subcore_barrier()

    pltpu.sync_copy(dst_hbm.at[pl.ds(base, NJ)], idx2)
    pltpu.async_copy(msg_hbm.at[pl.ds(base * CHUNK, CHUNK)], vals0, semv0)

    def step(u, carry):
        for b in range(2):
            t = 2 * u + b
            pltpu.make_async_copy(
                msg_hbm.at[pl.ds((base + t) * CHUNK, CHUNK)], vals[b],
                semv[b]).wait()
            pltpu.async_copy(vals[b], acc_s.at[idx2.at[t]], sems[b], add=True)
            if do_cnt:
                pltpu.sync_copy(ones_v, cnt_s.at[idx2.at[t]], add=True)

            @pl.when(t + 1 < NJ)
            def _():
                @pl.when(t >= 1)
                def _():
                    pltpu.make_async_copy(
                        vals[1 - b], acc_s.at[idx2.at[t - 1]],
                        sems[1 - b]).wait()
                pltpu.async_copy(
                    msg_hbm.at[pl.ds((base + t + 1) * CHUNK, CHUNK)],
                    vals[1 - b], semv[1 - b])

        return carry

    lax.fori_loop(0, NJ // 2, step, 0)
    pltpu.make_async_copy(vals0, acc_s.at[idx2.at[NJ - 2]], sems0).wait()
    pltpu.make_async_copy(vals1, acc_s.at[idx2.at[NJ - 1]], sems1).wait()
    plsc.subcore_barrier()

    @pl.when(sid < nslab)
    def _():
        rows = pl.ds(sid * ZROWS, ZROWS)
        pltpu.sync_copy(acc_s.at[rows], parts_hbm.at[cid].at[rows])

    if do_cnt:
        ncs = cnt_s.shape[0] // 1024

        @pl.when(sid == nslab)
        def _():
            for t in range(ncs):
                rows = pl.ds(t * 1024, 1024)
                pltpu.sync_copy(cnt_s.at[rows], cnt_hbm.at[cid].at[rows])


def _sc_scatter(msg, dst2, n, do_cnt):
    e, f_out = msg.shape
    npad = ((n + DUMP + 1023) // 1024) * 1024
    zero = jnp.zeros((ZROWS, f_out), jnp.float32)
    zero1 = jnp.zeros((1024,), jnp.float32)
    out_type = [jax.ShapeDtypeStruct((NC, n, f_out), jnp.float32)]
    scratch = [
        pltpu.VMEM((NJ, CHUNK), jnp.int32),
        pltpu.VMEM((CHUNK, f_out), jnp.float32),
        pltpu.VMEM((CHUNK, f_out), jnp.float32),
    ]
    if do_cnt:
        out_type.append(jax.ShapeDtypeStruct((NC, npad), jnp.float32))
        scratch.append(pltpu.VMEM((CHUNK,), jnp.float32))
    scratch += [
        pltpu.SemaphoreType.DMA,
        pltpu.SemaphoreType.DMA,
        pltpu.SemaphoreType.DMA,
        pltpu.SemaphoreType.DMA,
    ]
    scratch.append(pltpu.VMEM_SHARED((n + DUMP, f_out), jnp.float32))
    if do_cnt:
        scratch.append(pltpu.VMEM_SHARED((npad,), jnp.float32))
    k = pl.kernel(
        functools.partial(_scatter_body, n, do_cnt),
        out_type=tuple(out_type) if do_cnt else out_type[0],
        mesh=_sc_mesh(),
        scratch_types=scratch,
    )
    return k(msg, dst2, zero, zero1)


def _edge_body(attr_ref, xe_ref, mu_ref, alpha_ref, g_ref, msg_ref, z_ref):
    eb = attr_ref.shape[0]
    acc_g = jnp.zeros((eb, KK), dtype=jnp.float32)
    for d in range(DD):
        col = attr_ref[:, d : d + 1]
        diff = col - mu_ref[d : d + 1, :]
        acc_g = acc_g + diff * diff * alpha_ref[d : d + 1, :]
    gauss = jnp.exp(acc_g).astype(jnp.bfloat16)
    xe = xe_ref[...].astype(jnp.bfloat16)
    for k in range(KK):
        z_ref[:, k * FW : (k + 1) * FW] = xe * gauss[:, k : k + 1]
    msg_ref[...] = jnp.dot(z_ref[...], g_ref[...],
                           preferred_element_type=jnp.float32)


def _edge_msgs(edge_attr, xe, mu_t, alpha_t, g3):
    e = xe.shape[0]
    f_in = xe.shape[1]
    f_out = g3.shape[2]
    g_flat = g3.reshape(KK * f_in, f_out).astype(jnp.bfloat16)
    assert e % EDGE_BLK == 0
    grid = e // EDGE_BLK
    return pl.pallas_call(
        _edge_body,
        grid=(grid,),
        in_specs=[
            pl.BlockSpec((EDGE_BLK, DD), lambda i: (i, 0)),
            pl.BlockSpec((EDGE_BLK, f_in), lambda i: (i, 0)),
            pl.BlockSpec((DD, KK), lambda i: (0, 0)),
            pl.BlockSpec((DD, KK), lambda i: (0, 0)),
            pl.BlockSpec((KK * f_in, f_out), lambda i: (0, 0)),
        ],
        out_specs=pl.BlockSpec((EDGE_BLK, f_out), lambda i: (i, 0)),
        out_shape=jax.ShapeDtypeStruct((e, f_out), jnp.float32),
        scratch_shapes=[pltpu.VMEM((EDGE_BLK, KK * f_in), jnp.bfloat16)],
    )(edge_attr, xe, mu_t, alpha_t, g_flat)


def _node_body(do_act, parts_ref, cnt_ref, x_ref, root_ref, bias_ref,
               gamma_ref, beta_ref, out_ref):
    agg = (parts_ref[0] + parts_ref[1]) / jnp.maximum(cnt_ref[...], 1.0)
    r = jnp.dot(x_ref[...], root_ref[...], preferred_element_type=jnp.float32)
    h = agg + r + bias_ref[...]
    if do_act:
        h = jnp.where(h > 0, h, jnp.exp(h) - 1.0)
        m = jnp.mean(h, axis=0, keepdims=True)
        c = h - m
        v = jnp.mean(c * c, axis=0, keepdims=True)
        h = c / jnp.sqrt(v + 1e-5) * gamma_ref[...] + beta_ref[...]
    out_ref[...] = h


def _node_update(parts, cnt, x, root, bias, gamma, beta, do_act):
    n = x.shape[0]
    f_out = root.shape[1]
    return pl.pallas_call(
        functools.partial(_node_body, do_act),
        out_shape=jax.ShapeDtypeStruct((n, f_out), jnp.float32),
    )(parts, cnt, x, root, bias, gamma, beta)


def _padw(a, w=FW):
    pad = [(0, 0)] * (a.ndim - 1) + [(0, w - a.shape[-1])]
    return jnp.pad(a, pad)


def kernel(x, edge_index, edge_attr, params):
    n = x.shape[0]
    e = edge_attr.shape[0]
    f_final = params["conv4"]["root"].shape[1]
    assert e <= EP
    src = edge_index[0].astype(jnp.int32)
    dst = edge_index[1].astype(jnp.int32)
    src_pad = jnp.arange(EP - e, dtype=jnp.int32) % n
    src2 = jnp.concatenate([src, src_pad]).reshape(EP // CHUNK, CHUNK)
    dst_pad = n + jnp.arange(EP - e, dtype=jnp.int32) % DUMP
    dst2 = jnp.concatenate([dst, dst_pad]).reshape(EP // CHUNK, CHUNK)
    attr_p = jnp.pad(edge_attr, ((0, EP - e), (0, 0)))
    h = _padw(x)
    cnt = None
    names = ("conv1", "conv2", "conv3", "conv4")
    bns = ("bn1", "bn2", "bn3", None)
    for name, bn in zip(names, bns):
        p = params[name]
        f_in, f_out = p["root"].shape
        mu_t = p["mu"].T
        alpha_t = (-0.5 / (EPSG + p["sigma"] ** 2)).T
        g3 = p["g"].reshape(f_in, KK, f_out).transpose(1, 0, 2)
        g3 = jnp.pad(g3, ((0, 0), (0, FW - f_in), (0, FW - f_out)))
        xe = _sc_gather(h, src2)
        msg = _edge_msgs(attr_p, xe, mu_t, alpha_t, g3)
        if cnt is None:
            parts, cnt2 = _sc_scatter(msg, dst2, n, do_cnt=True)
            cnt = (cnt2[0, :n] + cnt2[1, :n]).reshape(n, 1)
        else:
            parts = _sc_scatter(msg, dst2, n, do_cnt=False)
        if bn is None:
            gamma = jnp.ones((1, FW), jnp.float32)
            beta = jnp.zeros((1, FW), jnp.float32)
        else:
            gamma = _padw(params[bn]["gamma"].reshape(1, f_out))
            beta = _padw(params[bn]["beta"].reshape(1, f_out))
        h = _node_update(parts, cnt, h, jnp.pad(p["root"], ((0, FW - f_in), (0, FW - f_out))),
                         _padw(p["bias"].reshape(1, f_out)), gamma, beta,
                         do_act=bn is not None)
    return h[:, :f_final]
